# Initial kernel scaffold; baseline (speedup 1.0000x reference)
import jax
import jax.numpy as jnp
from jax.experimental import pallas as pl


def kernel(A, all_neighbors):
    raise NotImplementedError("write your pallas kernel here")


# speedup vs baseline: 28.5147x; 28.5147x over previous
import functools

import jax
import jax.numpy as jnp
from jax import lax
from jax.experimental import pallas as pl
from jax.experimental.pallas import tpu as pltpu
from jax.experimental.pallas import tpu_sc as plsc

N_NODES = 100000
N_PAIRS = 3200000
NC = 2
NS = 16
NW = NC * NS

PAIRS_PER_W = N_PAIRS // NW
C = 1000
NCHUNK = PAIRS_PER_W // C
ROWS = 2 * C
SUB = 80
NSUB = ROWS // SUB

_mesh = plsc.VectorSubcoreMesh(core_axis_name="c", subcore_axis_name="s")


@functools.partial(
    pl.kernel,
    mesh=_mesh,
    out_type=jax.ShapeDtypeStruct((NW, 16), jnp.float32),
    scratch_types=[
        pltpu.VMEM((ROWS,), jnp.int32),
        pltpu.VMEM((ROWS, 16), jnp.float32),
        pltpu.VMEM((16,), jnp.float32),
        pltpu.SemaphoreType.DMA,
    ],
    compiler_params=pltpu.CompilerParams(use_tc_tiling_on_sc=False),
)
def _smoothness_kernel(x_hbm, nbr_hbm, out_hbm, idx_v, rows_v, acc_v, sem):
    wid = lax.axis_index("s") * NC + lax.axis_index("c")
    base_row = wid * (2 * PAIRS_PER_W)

    def chunk_body(c_i, acc):
        off = pl.multiple_of(base_row + c_i * ROWS, 8)
        pltpu.sync_copy(nbr_hbm.at[pl.ds(off, ROWS)], idx_v)
        handles = []
        for j in range(NSUB):
            handles.append(pltpu.async_copy(
                x_hbm.at[idx_v.at[pl.ds(j * SUB, SUB)]],
                rows_v.at[pl.ds(j * SUB, SUB)],
                sem,
            ))
        for h in handles:
            h.wait()

        def pair_body(k, a):
            r0 = rows_v[2 * k]
            r1 = rows_v[2 * k + 1]
            d = r0 - r1
            return a + d * d

        return lax.fori_loop(0, C, pair_body, acc, unroll=8)

    acc = lax.fori_loop(0, NCHUNK, chunk_body,
                        jnp.zeros((16,), jnp.float32))
    acc_v[...] = acc
    pltpu.sync_copy(acc_v, out_hbm.at[wid])


def kernel(A, all_neighbors):
    x = A.reshape(N_NODES, 16)
    nbr = all_neighbors.reshape(-1)
    partial = _smoothness_kernel(x, nbr)
    return jnp.sum(partial)


# speedup vs baseline: 29.9365x; 1.0499x over previous
import functools

import jax
import jax.numpy as jnp
from jax import lax
from jax.experimental import pallas as pl
from jax.experimental.pallas import tpu as pltpu
from jax.experimental.pallas import tpu_sc as plsc

N_NODES = 100000
N_PAIRS = 3200000
NC = 2
NS = 16
NW = NC * NS

PAIRS_PER_W = N_PAIRS // NW
C = 1000
NCHUNK = PAIRS_PER_W // C
ROWS = 2 * C
SUB = 80
NSUB = ROWS // SUB

_mesh = plsc.VectorSubcoreMesh(core_axis_name="c", subcore_axis_name="s")


@functools.partial(
    pl.kernel,
    mesh=_mesh,
    out_type=jax.ShapeDtypeStruct((NW, 16), jnp.float32),
    scratch_types=[
        pltpu.VMEM((2, ROWS), jnp.int32),
        pltpu.VMEM((2, ROWS, 16), jnp.float32),
        pltpu.VMEM((16,), jnp.float32),
        pltpu.SemaphoreType.DMA,
        pltpu.SemaphoreType.DMA,
    ],
    compiler_params=pltpu.CompilerParams(use_tc_tiling_on_sc=False),
)
def _smoothness_kernel(x_hbm, nbr_hbm, out_hbm, idx_v, rows_v, acc_v,
                       sem0, sem1):
    wid = lax.axis_index("s") * NC + lax.axis_index("c")
    base_row = wid * (2 * PAIRS_PER_W)
    sems = (sem0, sem1)

    def fetch(c_i, b):
        off = pl.multiple_of(base_row + c_i * ROWS, 8)
        pltpu.sync_copy(nbr_hbm.at[pl.ds(off, ROWS)], idx_v.at[b])
        for j in range(NSUB):
            pltpu.async_copy(
                x_hbm.at[idx_v.at[b, pl.ds(j * SUB, SUB)]],
                rows_v.at[b, pl.ds(j * SUB, SUB)],
                sems[b],
            )

    def drain(b):
        for j in range(NSUB):
            pltpu.make_async_copy(
                x_hbm.at[idx_v.at[b, pl.ds(j * SUB, SUB)]],
                rows_v.at[b, pl.ds(j * SUB, SUB)],
                sems[b],
            ).wait()

    fetch(0, 0)

    def step(t, acc):
        for b in (0, 1):
            c_i = 2 * t + b

            @pl.when(c_i + 1 < NCHUNK)
            def _():
                fetch(c_i + 1, 1 - b)

            drain(b)

            def pair_body(k, a):
                r0 = rows_v[b, 2 * k]
                r1 = rows_v[b, 2 * k + 1]
                d = r0 - r1
                return a + d * d

            acc = lax.fori_loop(0, C, pair_body, acc, unroll=8)
        return acc

    acc = lax.fori_loop(0, NCHUNK // 2, step,
                        jnp.zeros((16,), jnp.float32))
    acc_v[...] = acc
    pltpu.sync_copy(acc_v, out_hbm.at[wid])


def kernel(A, all_neighbors):
    x = A.reshape(N_NODES, 16)
    nbr = all_neighbors.reshape(-1)
    partial = _smoothness_kernel(x, nbr)
    return jnp.sum(partial)
